# baseline (speedup 1.0000x reference)
import jax
import jax.numpy as jnp
from jax import lax
from jax.experimental import pallas as pl
from jax.experimental.pallas import tpu as pltpu
from jax.experimental.pallas import tpu_sc as plsc

_V = 1000000
_EMB = 64
_L = 200
_SCALE = 8.0

_NC = 2
_NS = 16
_NW = _NC * _NS
_BB = 128

_FULLC = _V // 128
_PACKR = (_V + 64) // 2


def _prep_body(tt_hbm, tail_hbm, out_hbm, ib, ob, isem, osem):
    wid = lax.axis_index("s") * _NC + lax.axis_index("c")
    base = wid * 244 + jnp.minimum(wid, 4)
    ncols = 244 + jnp.where(wid < 4, 1, 0)

    rows16 = [lax.iota(jnp.int32, 16) + (g * 16) for g in range(4)]

    def start_in(c, p):
        pltpu.async_copy(tt_hbm.at[:, pl.ds(c * 128, 128)], ib[p], isem[p])

    def wait_in(p):
        pltpu.make_async_copy(tt_hbm.at[:, pl.ds(0, 128)], ib[p],
                              isem[p]).wait()

    def start_out(c, p):
        pltpu.async_copy(ob[p], out_hbm.at[pl.ds(c * 64, 64)], osem[p])

    def wait_out(p):
        pltpu.make_async_copy(ob[p], out_hbm.at[pl.ds(0, 64)],
                              osem[p]).wait()

    def transpose(p):
        src, dst = ib[p], ob[p]

        def rloop(r, carry):
            for h in range(2):
                jv = jnp.full((16,), 2 * r + h, jnp.int32)
                vs = [plsc.load_gather(src, [rows16[g], jv]) for g in range(4)]
                for g in range(4):
                    dst[r, pl.ds(h * 64 + g * 16, 16)] = vs[g] * _SCALE
            return carry

        lax.fori_loop(0, 64, rloop, 0)

    start_in(base, 0)

    def body(k, carry):
        p = lax.rem(k, 2)

        @pl.when(k + 1 < ncols)
        def _():
            for q in range(2):
                @pl.when(lax.rem(k + 1, 2) == q)
                def _():
                    start_in(base + k + 1, q)
        for q in range(2):
            @pl.when(p == q)
            def _():
                wait_in(q)

                @pl.when(k >= 2)
                def _():
                    wait_out(q)
                transpose(q)
                start_out(base + k, q)
        return carry

    lax.fori_loop(0, ncols, body, 0)

    @pl.when(wid == _NW - 1)
    def _():
        wait_out(0)
        pltpu.sync_copy(tail_hbm, ib[0])

        def tloop(r, carry):
            for h in range(2):
                for g in range(4):
                    ob[0][r, pl.ds(h * 64 + g * 16, 16)] = \
                        ib[0][2 * r + h, pl.ds(g * 16, 16)] * _SCALE
            return carry

        lax.fori_loop(0, 32, tloop, 0)
        pltpu.sync_copy(ob[0].at[pl.ds(0, 32)],
                        out_hbm.at[pl.ds(_FULLC * 64, 32)])
    wait_out(0)
    wait_out(1)


_CL = 4


def _emb_body(tokens_hbm, table_hbm, out_hbm, idx_v, gbufs, tbufs,
              gsems, osems):
    wid = lax.axis_index("s") * _NC + lax.axis_index("c")

    pltpu.sync_copy(tokens_hbm.at[wid], idx_v)

    def dbl(i, c):
        for j in range(_BB // 16):
            sl = pl.ds(j * 16, 16)
            idx_v[i, sl] = idx_v[i, sl] * 2
        return c

    lax.fori_loop(0, _L, dbl, 0)

    def start_gathers(g, p):
        for j in range(_CL):
            pltpu.async_copy(table_hbm.at[idx_v.at[g * _CL + j]],
                             gbufs[p].at[pl.ds(j * _BB, _BB)], gsems[p])

    def wait_gathers(p):
        pltpu.make_async_copy(table_hbm.at[pl.ds(0, _CL * _BB)], gbufs[p],
                              gsems[p]).wait()

    def start_out(l, p):
        pltpu.async_copy(tbufs[p], out_hbm.at[l, :, wid], osems[p])

    def wait_out(p):
        pltpu.make_async_copy(tbufs[p], out_hbm.at[0, :, wid],
                              osems[p]).wait()

    rows = [lax.iota(jnp.int32, 16) + (bj * 16) for bj in range(_BB // 16)]

    def transpose(gp, j, tp):
        src, dst = gbufs[gp], tbufs[tp]
        roff = j * _BB

        def col(e8, c):
            for ee in range(8):
                ev = jnp.full((16,), e8 * 8 + ee, jnp.int32)
                vs = [plsc.load_gather(src, [rows[bj] + roff, ev])
                      for bj in range(_BB // 16)]
                for bj in range(_BB // 16):
                    dst[e8, ee, pl.ds(bj * 16, 16)] = vs[bj] * _SCALE
            return c

        lax.fori_loop(0, _EMB // 8, col, 0)

    n_chunks = _L // _CL
    start_gathers(0, 0)

    def outer(g2, carry):
        g0 = 2 * g2
        for p in range(2):
            g = g0 + p
            other = 1 - p

            @pl.when(g + 1 < n_chunks)
            def _():
                start_gathers(g + 1, other)
            wait_gathers(p)
            for j in range(_CL):
                tp = j % 2

                @pl.when(g > 0)
                def _():
                    wait_out(tp)
                transpose(p, j, tp)
                start_out(g * _CL + j, tp)
        return carry

    lax.fori_loop(0, n_chunks // 2, outer, 0)
    wait_out(0)
    wait_out(1)


def kernel(tokens, table):
    b, l = tokens.shape
    mesh = plsc.VectorSubcoreMesh(core_axis_name="c", subcore_axis_name="s")

    table_lin = jnp.pad(table, ((0, 0), (0, 64))).reshape(-1, _EMB)

    tokens_arr = tokens.T.reshape(l, _NW, _BB).transpose(1, 0, 2)
    out5 = pl.kernel(
        _emb_body,
        out_type=jax.ShapeDtypeStruct((l, _EMB // 8, _NW, 8, _BB),
                                      jnp.float32),
        mesh=mesh,
        scratch_types=[
            pltpu.VMEM((_L, _BB), jnp.int32),
            [pltpu.VMEM((_CL * _BB, _EMB), jnp.float32) for _ in range(2)],
            [pltpu.VMEM((_EMB // 8, 8, _BB), jnp.float32) for _ in range(2)],
            [pltpu.SemaphoreType.DMA for _ in range(2)],
            [pltpu.SemaphoreType.DMA for _ in range(2)],
        ],
        compiler_params=pltpu.CompilerParams(use_tc_tiling_on_sc=False,
                                             needs_layout_passes=False),
    )(tokens_arr, table_lin)
    return out5.transpose(2, 4, 0, 1, 3).reshape(b, l, _EMB)


# speedup vs baseline: 1.2615x; 1.2615x over previous
import jax
import jax.numpy as jnp
from jax import lax
from jax.experimental import pallas as pl
from jax.experimental.pallas import tpu as pltpu
from jax.experimental.pallas import tpu_sc as plsc

_EMB = 64
_SCALE = 8.0

_NC = 2
_NS = 16
_NW = _NC * _NS

_CHUNK = 640
_DMA_ROWS = 128
_K = _CHUNK // _DMA_ROWS


def _emb_body(tokens_hbm, table_hbm, out_hbm,
              idx_all, rows0, rows1, gsem0, gsem1, osem0, osem1):
    n_tok = tokens_hbm.shape[0]
    per_w = n_tok // _NW
    n_chunks = per_w // _CHUNK
    n2 = n_chunks // 2
    wid = lax.axis_index("s") * _NC + lax.axis_index("c")
    base = wid * per_w

    rows = (rows0, rows1)
    gsem = (gsem0, gsem1)
    osem = (osem0, osem1)

    pltpu.sync_copy(tokens_hbm.at[pl.ds(base, per_w)], idx_all)

    def start_gather(gg, b):
        for j in range(_K):
            pltpu.async_copy(
                table_hbm.at[idx_all.at[pl.ds(gg * _CHUNK + j * _DMA_ROWS,
                                              _DMA_ROWS)]],
                rows[b].at[pl.ds(j * _DMA_ROWS, _DMA_ROWS)],
                gsem[b],
            )

    def wait_gather(b):
        pltpu.make_async_copy(table_hbm.at[pl.ds(0, _CHUNK)], rows[b],
                              gsem[b]).wait()

    def start_outcopy(gg, b):
        pltpu.async_copy(rows[b], out_hbm.at[pl.ds(base + gg * _CHUNK, _CHUNK)],
                         osem[b])

    def wait_outcopy(b):
        pltpu.make_async_copy(rows[b], out_hbm.at[pl.ds(0, _CHUNK)],
                              osem[b]).wait()

    def scale(b):
        buf = rows[b]

        def srow(i, c):
            r = i * 4
            for dr in range(4):
                for c4 in range(_EMB // 16):
                    sl = pl.ds(c4 * 16, 16)
                    buf[r + dr, sl] = buf[r + dr, sl] * _SCALE
            return c

        lax.fori_loop(0, _CHUNK // 4, srow, 0)

    start_gather(0, 0)

    def outer(g2, carry):
        gg0 = 2 * g2

        @pl.when(g2 > 0)
        def _():
            wait_outcopy(1)
        start_gather(gg0 + 1, 1)
        wait_gather(0)
        scale(0)
        start_outcopy(gg0, 0)

        @pl.when(g2 < n2 - 1)
        def _():
            wait_outcopy(0)
            start_gather(gg0 + 2, 0)
        wait_gather(1)
        scale(1)
        start_outcopy(gg0 + 1, 1)
        return carry

    lax.fori_loop(0, n2, outer, 0)
    wait_outcopy(0)
    wait_outcopy(1)


def kernel(tokens, table):
    b, l = tokens.shape
    n_tok = b * l
    flat = tokens.reshape(n_tok)
    per_w = n_tok // _NW
    mesh = plsc.VectorSubcoreMesh(core_axis_name="c", subcore_axis_name="s")
    out = pl.kernel(
        _emb_body,
        out_type=jax.ShapeDtypeStruct((n_tok, _EMB), jnp.float32),
        mesh=mesh,
        scratch_types=[
            pltpu.VMEM((per_w,), jnp.int32),
            pltpu.VMEM((_CHUNK, _EMB), jnp.float32),
            pltpu.VMEM((_CHUNK, _EMB), jnp.float32),
            pltpu.SemaphoreType.DMA,
            pltpu.SemaphoreType.DMA,
            pltpu.SemaphoreType.DMA,
            pltpu.SemaphoreType.DMA,
        ],
        compiler_params=pltpu.CompilerParams(use_tc_tiling_on_sc=False),
    )(flat, table)
    return out.reshape(b, l, _EMB)
